# baseline (speedup 1.0000x reference)
import jax
import jax.numpy as jnp
from jax import lax
from jax.experimental import pallas as pl
from jax.experimental.pallas import tpu as pltpu
from jax.experimental.pallas import tpu_sc as plsc

B, F, V = 16384, 26, 1000000
NC, NS = 2, 16
NW = NC * NS
E = (B * F) // NW
BLK = 1024


def _sc_gather_body(idx_hbm, w_hbm, g_hbm, idx_v, g_v, sem):
    wid = lax.axis_index("s") * NC + lax.axis_index("c")
    base = wid * E
    pltpu.sync_copy(idx_hbm.at[pl.ds(base, E)], idx_v)
    pltpu.async_copy(w_hbm.at[idx_v], g_v, sem).wait()
    pltpu.sync_copy(g_v, g_hbm.at[pl.ds(base, E)])


def _tc_reduce_body(b_ref, g_ref, v_ref, o_ref):
    x = jnp.sum(g_ref[...] * v_ref[...], axis=1) + b_ref[0]
    o_ref[...] = jax.nn.sigmoid(x)


def kernel(feature_idx, feature_vals, W, b):
    idx_flat = feature_idx.astype(jnp.int32).reshape(B * F)

    mesh = plsc.VectorSubcoreMesh(core_axis_name="c", subcore_axis_name="s")
    gathered = pl.kernel(
        _sc_gather_body,
        out_type=jax.ShapeDtypeStruct((B * F,), jnp.float32),
        mesh=mesh,
        scratch_types=[
            pltpu.VMEM((E,), jnp.int32),
            pltpu.VMEM((E,), jnp.float32),
            pltpu.SemaphoreType.DMA,
        ],
    )(idx_flat, W)

    return pl.pallas_call(
        _tc_reduce_body,
        grid=(B // BLK,),
        in_specs=[
            pl.BlockSpec(memory_space=pltpu.SMEM),
            pl.BlockSpec((BLK, F), lambda i: (i, 0)),
            pl.BlockSpec((BLK, F), lambda i: (i, 0)),
        ],
        out_specs=pl.BlockSpec((BLK,), lambda i: (i,)),
        out_shape=jax.ShapeDtypeStruct((B,), jnp.float32),
    )(jnp.asarray(b, jnp.float32).reshape(1),
      gathered.reshape(B, F), feature_vals)


# speedup vs baseline: 1.2975x; 1.2975x over previous
import jax
import jax.numpy as jnp
from jax import lax
from jax.experimental import pallas as pl
from jax.experimental.pallas import tpu as pltpu
from jax.experimental.pallas import tpu_sc as plsc

B, F, V = 16384, 26, 1000000
NC, NS = 2, 16
NW = NC * NS
E = (B * F) // NW

XR = (B * F) // 128
NBLK = 8
BR = XR // NBLK
SUP = 13
NSUP = BR // SUP
SEG = 64


def _sc_gather_body(idx_hbm, w_hbm, g_hbm, idx_v, g_v, sem):
    wid = lax.axis_index("s") * NC + lax.axis_index("c")
    base = wid * E
    pltpu.sync_copy(idx_hbm.at[pl.ds(base, E)], idx_v)
    pltpu.async_copy(w_hbm.at[idx_v], g_v, sem).wait()
    pltpu.sync_copy(g_v, g_hbm.at[pl.ds(base, E)])


def _tc_reduce_body(b_ref, g_ref, v_ref, m_ref, o_ref):
    p = (g_ref[...] * v_ref[...]).reshape(NSUP, SUP, 128)
    acc = jnp.zeros((NSUP, SEG), jnp.float32)
    for r in range(SUP):
        acc = acc + jnp.dot(p[:, r, :], m_ref[r],
                            preferred_element_type=jnp.float32)
    o_ref[...] = jax.nn.sigmoid(acc + b_ref[0])


def kernel(feature_idx, feature_vals, W, b):
    idx_flat = feature_idx.astype(jnp.int32).reshape(B * F)

    mesh = plsc.VectorSubcoreMesh(core_axis_name="c", subcore_axis_name="s")
    gathered = pl.kernel(
        _sc_gather_body,
        out_type=jax.ShapeDtypeStruct((B * F,), jnp.float32),
        mesh=mesh,
        scratch_types=[
            pltpu.VMEM((E,), jnp.int32),
            pltpu.VMEM((E,), jnp.float32),
            pltpu.SemaphoreType.DMA,
        ],
    )(idx_flat, W)

    r = jnp.arange(SUP, dtype=jnp.int32)[:, None, None]
    c = jnp.arange(128, dtype=jnp.int32)[None, :, None]
    s = jnp.arange(SEG, dtype=jnp.int32)[None, None, :]
    flat = 128 * r + c
    m = ((flat >= F * s) & (flat < F * s + F)).astype(jnp.float32)

    out2d = pl.pallas_call(
        _tc_reduce_body,
        grid=(NBLK,),
        in_specs=[
            pl.BlockSpec(memory_space=pltpu.SMEM),
            pl.BlockSpec((BR, 128), lambda i: (i, 0)),
            pl.BlockSpec((BR, 128), lambda i: (i, 0)),
            pl.BlockSpec((SUP, 128, SEG), lambda i: (0, 0, 0)),
        ],
        out_specs=pl.BlockSpec((NSUP, SEG), lambda i: (i, 0)),
        out_shape=jax.ShapeDtypeStruct((NBLK * NSUP, SEG), jnp.float32),
    )(jnp.asarray(b, jnp.float32).reshape(1),
      gathered.reshape(XR, 128), feature_vals.reshape(XR, 128), m)
    return out2d.reshape(B)
